# baseline (speedup 1.0000x reference)
import numpy as np

import jax
import jax.numpy as jnp
from jax import lax
from jax.experimental import pallas as pl
from jax.experimental.pallas import tpu as pltpu
from jax.experimental.pallas import tpu_sc as plsc

N, K, H, W, C, P = 4, 8, 256, 256, 64, 100000
HW = H * W
NP = N * HW
LANES = 16

NUM_CORES = 2
NUM_SUBCORES = 16
NW = NUM_CORES * NUM_SUBCORES
PPW = NP // NW
B = 64
ROWS = K * B
CV = C // LANES
SLAB = 1024
CPS = SLAB // B
NSLAB = PPW // SLAB
NPAIRS = PPW // B // 2
PPSLAB = CPS // 2
UNROLL = 4
CW = C // 32

_PERM = np.empty((C,), np.int64)
for _j in range(CW):
    for _l in range(LANES):
        _PERM[32 * _j + 2 * _l] = 32 * _j + _l
        _PERM[32 * _j + 2 * _l + 1] = 32 * _j + LANES + _l


def _sc_body(frag_hbm, alpha_hbm, table_hbm, out_hbm,
             frag_v, alpha_v, rows_v, out_v,
             sem_in, sem_g0, sem_g1, sem_o0, sem_o1):
    cid = lax.axis_index("c")
    sid = lax.axis_index("s")
    wid = sid * NUM_CORES + cid
    pbase = wid * PPW
    n = pbase // HW
    off0 = pbase % HW
    row0 = n * K

    def issue_inputs(s, p):
        pltpu.async_copy(
            frag_hbm.at[pl.ds((pbase + s * SLAB) * K, SLAB * K)],
            frag_v.at[p], sem_in)
        for k in range(K):
            src = pl.ds(off0 + s * SLAB, SLAB)
            pltpu.async_copy(alpha_hbm.at[row0 + k, src],
                             alpha_v.at[p, k], sem_in)

    def wait_inputs(p):
        pltpu.make_async_copy(frag_hbm.at[pl.ds(0, SLAB * K)],
                              frag_v.at[p], sem_in).wait()
        pltpu.make_async_copy(alpha_hbm.at[pl.ds(0, K), pl.ds(0, SLAB)],
                              alpha_v.at[p], sem_in).wait()

    def issue_gathers(c, gp, sem):
        p = (c // CPS) % 2
        ioff = (c % CPS) * ROWS
        for g in range(ROWS // 128):
            pltpu.async_copy(
                table_hbm.at[frag_v.at[p, pl.ds(ioff + g * 128, 128)]],
                rows_v.at[gp, pl.ds(g * 128, 128)], sem)

    def wait_gathers(gp, sem):
        pltpu.make_async_copy(table_hbm.at[pl.ds(0, ROWS)],
                              rows_v.at[gp], sem).wait()

    mask_hi = jnp.full((LANES,), -65536, jnp.int32)

    def wait_out(op, sem):
        pltpu.make_async_copy(table_hbm.at[pl.ds(0, B)],
                              out_v.at[op], sem).wait()

    def compute_chunk(c, gp):
        p = (c // CPS) % 2
        ioff = (c % CPS) * B
        one = jnp.ones((LANES,), jnp.float32)
        for g in range(B // LANES):
            wk = []
            run = one
            for k in range(K):
                a = alpha_v[p, k, pl.ds(ioff + g * LANES, LANES)]
                wk.append(a * run)
                run = run * (one - a)

            def lane_body(lv, wks):
                for dl in range(UNROLL):
                    l = lv * UNROLL + dl
                    b = g * LANES + l
                    lane = jnp.full((LANES,), l, jnp.int32)
                    accs = [None] * CV
                    for k in range(K):
                        ws = wks[k].at[lane].get(mode="promise_in_bounds")
                        for cw in range(CW):
                            v = rows_v[gp, b * K + k,
                                       pl.ds(cw * LANES, LANES)]
                            lo = lax.bitcast_convert_type(
                                lax.shift_left(v, 16), jnp.float32)
                            hi = lax.bitcast_convert_type(v, jnp.float32)
                            for cv, r in ((2 * cw, lo), (2 * cw + 1, hi)):
                                t = r * ws
                                accs[cv] = (t if accs[cv] is None
                                            else accs[cv] + t)
                    for cv in range(CV):
                        out_v[gp, b, pl.ds(cv * LANES, LANES)] = accs[cv]
                return wks

            lax.fori_loop(0, LANES // UNROLL, lane_body, tuple(wk))

    issue_inputs(0, 0)
    wait_inputs(0)
    issue_gathers(0, 0, sem_g0)

    def pair_body(j, carry):
        s = j // PPSLAB
        a = 2 * j

        @pl.when(jnp.logical_and(j % PPSLAB == 0, j < (NSLAB - 1) * PPSLAB))
        def _():
            issue_inputs(s + 1, (s + 1) % 2)

        issue_gathers(a + 1, 1, sem_g1)
        wait_gathers(0, sem_g0)

        @pl.when(j > 0)
        def _():
            wait_out(0, sem_o0)

        compute_chunk(a, 0)
        pltpu.async_copy(out_v.at[0], out_hbm.at[pl.ds(pbase + a * B, B)],
                         sem_o0)

        @pl.when(jnp.logical_and(j % PPSLAB == PPSLAB - 1, j < NPAIRS - 1))
        def _():
            wait_inputs((s + 1) % 2)

        @pl.when(j < NPAIRS - 1)
        def _():
            issue_gathers(a + 2, 0, sem_g0)

        wait_gathers(1, sem_g1)

        @pl.when(j > 0)
        def _():
            wait_out(1, sem_o1)

        compute_chunk(a + 1, 1)
        pltpu.async_copy(out_v.at[1],
                         out_hbm.at[pl.ds(pbase + (a + 1) * B, B)], sem_o1)
        return carry

    lax.fori_loop(0, NPAIRS, pair_body, 0)
    wait_out(0, sem_o0)
    wait_out(1, sem_o1)


@jax.jit
def _sc_call(frag2d, alpha2d, table):
    mesh = plsc.VectorSubcoreMesh(core_axis_name="c", subcore_axis_name="s")
    f = pl.kernel(
        _sc_body,
        out_type=jax.ShapeDtypeStruct((NP, C), jnp.float32),
        mesh=mesh,
        scratch_types=[
            pltpu.VMEM((2, K * SLAB), jnp.int32),
            pltpu.VMEM((2, K, SLAB), jnp.float32),
            pltpu.VMEM((2, ROWS, C // 2), jnp.int32),
            pltpu.VMEM((2, B, C), jnp.float32),
            pltpu.SemaphoreType.DMA,
            pltpu.SemaphoreType.DMA,
            pltpu.SemaphoreType.DMA,
            pltpu.SemaphoreType.DMA,
            pltpu.SemaphoreType.DMA,
        ],
        compiler_params=pltpu.CompilerParams(use_tc_tiling_on_sc=False),
    )
    return f(frag2d, alpha2d, table)


def kernel(fragments, alphas, ptclds):
    frag_il = fragments.astype(jnp.int32).transpose(0, 2, 3, 1).reshape(NP * K)
    alpha2d = alphas.reshape(N * K, HW)
    table_bf = ptclds.T.reshape(P, C)[:, _PERM].astype(jnp.bfloat16)
    table = lax.bitcast_convert_type(table_bf.reshape(P, C // 2, 2),
                                     jnp.int32)
    out_flat = _sc_call(frag_il, alpha2d, table)
    return out_flat.reshape(N, HW, C).transpose(0, 2, 1).reshape(N, C, H, W)


# speedup vs baseline: 1.1163x; 1.1163x over previous
import numpy as np

import jax
import jax.numpy as jnp
from jax import lax
from jax.experimental import pallas as pl
from jax.experimental.pallas import tpu as pltpu
from jax.experimental.pallas import tpu_sc as plsc

N, K, H, W, C, P = 4, 8, 256, 256, 64, 100000
HW = H * W
NP = N * HW
LANES = 16

NUM_CORES = 2
NUM_SUBCORES = 16
NW = NUM_CORES * NUM_SUBCORES
PPW = NP // NW
B = 64
ROWS = K * B
CV = C // LANES
SLAB = 1024
CPS = SLAB // B
NSLAB = PPW // SLAB
NPAIRS = PPW // B // 2
PPSLAB = CPS // 2
UNROLL = 4
CW = C // 32

_PERM = np.empty((C,), np.int64)
for _j in range(CW):
    for _l in range(LANES):
        _PERM[32 * _j + 2 * _l] = 32 * _j + _l
        _PERM[32 * _j + 2 * _l + 1] = 32 * _j + LANES + _l


def _sc_body(frag_hbm, alpha_hbm, table_hbm, out_hbm,
             frag_v, alpha_v, rows_v, out_v,
             sem_in, sem_g0, sem_g1, sem_o0, sem_o1):
    cid = lax.axis_index("c")
    sid = lax.axis_index("s")
    wid = sid * NUM_CORES + cid
    pbase = wid * PPW
    n = pbase // HW
    off0 = pbase % HW
    row0 = n * K

    def issue_inputs(s, p):
        for k in range(K):
            src = pl.ds(off0 + s * SLAB, SLAB)
            pltpu.async_copy(frag_hbm.at[row0 + k, src],
                             frag_v.at[p, k], sem_in)
            pltpu.async_copy(alpha_hbm.at[row0 + k, src],
                             alpha_v.at[p, k], sem_in)

    def wait_inputs(p):
        pltpu.make_async_copy(frag_hbm.at[pl.ds(0, K), pl.ds(0, SLAB)],
                              frag_v.at[p], sem_in).wait()
        pltpu.make_async_copy(alpha_hbm.at[pl.ds(0, K), pl.ds(0, SLAB)],
                              alpha_v.at[p], sem_in).wait()

    def issue_gathers(c, gp, sem):
        p = (c // CPS) % 2
        ioff = (c % CPS) * B
        for k in range(K):
            pltpu.async_copy(
                table_hbm.at[frag_v.at[p, k, pl.ds(ioff, B)]],
                rows_v.at[gp, pl.ds(k * B, B)], sem)

    def wait_gathers(gp, sem):
        pltpu.make_async_copy(table_hbm.at[pl.ds(0, ROWS)],
                              rows_v.at[gp], sem).wait()

    def wait_out(op, sem):
        pltpu.make_async_copy(table_hbm.at[pl.ds(0, B)],
                              out_v.at[op], sem).wait()

    def compute_chunk(c, gp):
        p = (c // CPS) % 2
        ioff = (c % CPS) * B
        one = jnp.ones((LANES,), jnp.float32)
        for g in range(B // LANES):
            wk = []
            run = one
            for k in range(K):
                a = alpha_v[p, k, pl.ds(ioff + g * LANES, LANES)]
                wk.append(a * run)
                run = run * (one - a)

            def lane_body(lv, wks):
                for dl in range(UNROLL):
                    l = lv * UNROLL + dl
                    b = g * LANES + l
                    lane = jnp.full((LANES,), l, jnp.int32)
                    accs = [None] * CV
                    for k in range(K):
                        ws = wks[k].at[lane].get(mode="promise_in_bounds")
                        for cw in range(CW):
                            v = rows_v[gp, k * B + b,
                                       pl.ds(cw * LANES, LANES)]
                            lo = lax.bitcast_convert_type(
                                lax.shift_left(v, 16), jnp.float32)
                            hi = lax.bitcast_convert_type(v, jnp.float32)
                            for cv, r in ((2 * cw, lo), (2 * cw + 1, hi)):
                                t = r * ws
                                accs[cv] = (t if accs[cv] is None
                                            else accs[cv] + t)
                    for cv in range(CV):
                        out_v[gp, b, pl.ds(cv * LANES, LANES)] = accs[cv]
                return wks

            lax.fori_loop(0, LANES // UNROLL, lane_body, tuple(wk))

    issue_inputs(0, 0)
    wait_inputs(0)
    issue_gathers(0, 0, sem_g0)

    def pair_body(j, carry):
        s = j // PPSLAB
        a = 2 * j

        @pl.when(jnp.logical_and(j % PPSLAB == 0, j < (NSLAB - 1) * PPSLAB))
        def _():
            issue_inputs(s + 1, (s + 1) % 2)

        issue_gathers(a + 1, 1, sem_g1)
        wait_gathers(0, sem_g0)

        @pl.when(j > 0)
        def _():
            wait_out(0, sem_o0)

        compute_chunk(a, 0)
        pltpu.async_copy(out_v.at[0], out_hbm.at[pl.ds(pbase + a * B, B)],
                         sem_o0)

        @pl.when(jnp.logical_and(j % PPSLAB == PPSLAB - 1, j < NPAIRS - 1))
        def _():
            wait_inputs((s + 1) % 2)

        @pl.when(j < NPAIRS - 1)
        def _():
            issue_gathers(a + 2, 0, sem_g0)

        wait_gathers(1, sem_g1)

        @pl.when(j > 0)
        def _():
            wait_out(1, sem_o1)

        compute_chunk(a + 1, 1)
        pltpu.async_copy(out_v.at[1],
                         out_hbm.at[pl.ds(pbase + (a + 1) * B, B)], sem_o1)
        return carry

    lax.fori_loop(0, NPAIRS, pair_body, 0)
    wait_out(0, sem_o0)
    wait_out(1, sem_o1)


@jax.jit
def _sc_call(frag2d, alpha2d, table):
    mesh = plsc.VectorSubcoreMesh(core_axis_name="c", subcore_axis_name="s")
    f = pl.kernel(
        _sc_body,
        out_type=jax.ShapeDtypeStruct((NP, C), jnp.float32),
        mesh=mesh,
        scratch_types=[
            pltpu.VMEM((2, K, SLAB), jnp.int32),
            pltpu.VMEM((2, K, SLAB), jnp.float32),
            pltpu.VMEM((2, ROWS, C // 2), jnp.int32),
            pltpu.VMEM((2, B, C), jnp.float32),
            pltpu.SemaphoreType.DMA,
            pltpu.SemaphoreType.DMA,
            pltpu.SemaphoreType.DMA,
            pltpu.SemaphoreType.DMA,
            pltpu.SemaphoreType.DMA,
        ],
        compiler_params=pltpu.CompilerParams(use_tc_tiling_on_sc=False),
    )
    return f(frag2d, alpha2d, table)


def kernel(fragments, alphas, ptclds):
    frag2d = fragments.reshape(N * K, HW).astype(jnp.int32)
    alpha2d = alphas.reshape(N * K, HW)
    table_bf = ptclds.T.reshape(P, C)[:, _PERM].astype(jnp.bfloat16)
    table = lax.bitcast_convert_type(table_bf.reshape(P, C // 2, 2),
                                     jnp.int32)
    out_flat = _sc_call(frag2d, alpha2d, table)
    return out_flat.reshape(N, HW, C).transpose(0, 2, 1).reshape(N, C, H, W)


# speedup vs baseline: 1.3349x; 1.1959x over previous
import numpy as np

import jax
import jax.numpy as jnp
from jax import lax
from jax.experimental import pallas as pl
from jax.experimental.pallas import tpu as pltpu
from jax.experimental.pallas import tpu_sc as plsc

N, K, H, W, C, P = 4, 8, 256, 256, 64, 100000
HW = H * W
NP = N * HW
LANES = 16

NUM_CORES = 2
NUM_SUBCORES = 16
NW = NUM_CORES * NUM_SUBCORES
PPW = NP // NW
B = 64
ROWS = K * B
CV = C // LANES
SLAB = 1024
CPS = SLAB // B
NSLAB = PPW // SLAB
NPAIRS = PPW // B // 2
PPSLAB = CPS // 2
UNROLL = 8
CW = C // 32

_PERM = np.empty((C,), np.int64)
for _j in range(CW):
    for _l in range(LANES):
        _PERM[32 * _j + 2 * _l] = 32 * _j + _l
        _PERM[32 * _j + 2 * _l + 1] = 32 * _j + LANES + _l


def _sc_body(frag_hbm, alpha_hbm, table_hbm, out_hbm,
             frag_v, alpha_v, rows_v, out_v,
             sem_in, sem_g0, sem_g1, sem_o0, sem_o1):
    cid = lax.axis_index("c")
    sid = lax.axis_index("s")
    wid = sid * NUM_CORES + cid
    pbase = wid * PPW
    n = pbase // HW
    off0 = pbase % HW
    row0 = n * K

    def issue_inputs(s, p):
        for k in range(K):
            src = pl.ds(off0 + s * SLAB, SLAB)
            pltpu.async_copy(frag_hbm.at[row0 + k, src],
                             frag_v.at[p, k], sem_in)
            pltpu.async_copy(alpha_hbm.at[row0 + k, src],
                             alpha_v.at[p, k], sem_in)

    def wait_inputs(p):
        pltpu.make_async_copy(frag_hbm.at[pl.ds(0, K), pl.ds(0, SLAB)],
                              frag_v.at[p], sem_in).wait()
        pltpu.make_async_copy(alpha_hbm.at[pl.ds(0, K), pl.ds(0, SLAB)],
                              alpha_v.at[p], sem_in).wait()

    def issue_gathers(c, gp, sem):
        p = (c // CPS) % 2
        ioff = (c % CPS) * B
        for k in range(K):
            pltpu.async_copy(
                table_hbm.at[frag_v.at[p, k, pl.ds(ioff, B)]],
                rows_v.at[gp, pl.ds(k * B, B)], sem)

    def wait_gathers(gp, sem):
        pltpu.make_async_copy(table_hbm.at[pl.ds(0, ROWS)],
                              rows_v.at[gp], sem).wait()

    def wait_out(op, sem):
        pltpu.make_async_copy(table_hbm.at[pl.ds(0, B)],
                              out_v.at[op], sem).wait()

    def compute_chunk(c, gp):
        p = (c // CPS) % 2
        ioff = (c % CPS) * B
        one = jnp.ones((LANES,), jnp.float32)
        for g in range(B // LANES):
            wk = []
            run = one
            for k in range(K):
                a = alpha_v[p, k, pl.ds(ioff + g * LANES, LANES)]
                wk.append(a * run)
                run = run * (one - a)

            def lane_body(lv, wks):
                for dl in range(UNROLL):
                    l = lv * UNROLL + dl
                    b = g * LANES + l
                    lane = jnp.full((LANES,), l, jnp.int32)
                    accs = [None] * CV
                    for k in range(K):
                        ws = wks[k].at[lane].get(mode="promise_in_bounds")
                        for cv in range(CV):
                            r = rows_v[gp, k * B + b,
                                       pl.ds(cv * LANES, LANES)]
                            t = r * ws
                            accs[cv] = t if accs[cv] is None else accs[cv] + t
                    for cv in range(CV):
                        out_v[gp, b, pl.ds(cv * LANES, LANES)] = accs[cv]
                return wks

            lax.fori_loop(0, LANES // UNROLL, lane_body, tuple(wk))

    issue_inputs(0, 0)
    wait_inputs(0)
    issue_gathers(0, 0, sem_g0)

    def pair_body(j, carry):
        s = j // PPSLAB
        a = 2 * j

        @pl.when(jnp.logical_and(j % PPSLAB == 0, j < (NSLAB - 1) * PPSLAB))
        def _():
            issue_inputs(s + 1, (s + 1) % 2)

        issue_gathers(a + 1, 1, sem_g1)
        wait_gathers(0, sem_g0)

        @pl.when(j > 0)
        def _():
            wait_out(0, sem_o0)

        compute_chunk(a, 0)
        pltpu.async_copy(out_v.at[0], out_hbm.at[pl.ds(pbase + a * B, B)],
                         sem_o0)

        @pl.when(jnp.logical_and(j % PPSLAB == PPSLAB - 1, j < NPAIRS - 1))
        def _():
            wait_inputs((s + 1) % 2)

        @pl.when(j < NPAIRS - 1)
        def _():
            issue_gathers(a + 2, 0, sem_g0)

        wait_gathers(1, sem_g1)

        @pl.when(j > 0)
        def _():
            wait_out(1, sem_o1)

        compute_chunk(a + 1, 1)
        pltpu.async_copy(out_v.at[1],
                         out_hbm.at[pl.ds(pbase + (a + 1) * B, B)], sem_o1)
        return carry

    lax.fori_loop(0, NPAIRS, pair_body, 0)
    wait_out(0, sem_o0)
    wait_out(1, sem_o1)


@jax.jit
def _sc_call(frag2d, alpha2d, table):
    mesh = plsc.VectorSubcoreMesh(core_axis_name="c", subcore_axis_name="s")
    f = pl.kernel(
        _sc_body,
        out_type=jax.ShapeDtypeStruct((NP, C), jnp.float32),
        mesh=mesh,
        scratch_types=[
            pltpu.VMEM((2, K, SLAB), jnp.int32),
            pltpu.VMEM((2, K, SLAB), jnp.float32),
            pltpu.VMEM((2, ROWS, C), jnp.float32),
            pltpu.VMEM((2, B, C), jnp.float32),
            pltpu.SemaphoreType.DMA,
            pltpu.SemaphoreType.DMA,
            pltpu.SemaphoreType.DMA,
            pltpu.SemaphoreType.DMA,
            pltpu.SemaphoreType.DMA,
        ],
        compiler_params=pltpu.CompilerParams(use_tc_tiling_on_sc=False),
    )
    return f(frag2d, alpha2d, table)


def kernel(fragments, alphas, ptclds):
    frag2d = fragments.reshape(N * K, HW).astype(jnp.int32)
    alpha2d = alphas.reshape(N * K, HW)
    table = ptclds.T.reshape(P, C)
    out_flat = _sc_call(frag2d, alpha2d, table)
    return out_flat.reshape(N, HW, C).transpose(0, 2, 1).reshape(N, C, H, W)
